# baseline (speedup 1.0000x reference)
import functools

import jax
import jax.numpy as jnp
from jax import lax
from jax.experimental import pallas as pl
from jax.experimental.pallas import tpu as pltpu
from jax.experimental.pallas import tpu_sc as plsc

N = 8192
K = 8
NC = 2
NS = 16
L = 16
CHUNK = N // NS
NV = CHUNK // L
NCAND = NS * L

NEG = float("-inf")
BIG = 0x3FFFFFFF

_DNUMS = lax.GatherDimensionNumbers(
    offset_dims=(), collapsed_slice_dims=(0,), start_index_map=(0,))


def _shuf(v, idx):
    return lax.gather(v, idx[:, None], _DNUMS, (1,),
                      mode=lax.GatherScatterMode.PROMISE_IN_BOUNDS)


def _bfly(v, op, iota):
    for k in (1, 2, 4, 8):
        v = op(v, _shuf(v, iota ^ k))
    return v


def _body(x_hbm, w_hbm, top_hbm,
          x_v, work_v, e_v,
          cv_sh, ci_sh, sum_sh,
          cv_loc, ci_loc, sum_loc,
          row_v, row_i, row_s, row_t):
    c = lax.axis_index("c")
    s = lax.axis_index("s")
    base = s * CHUNK
    src = c * N + base
    iota = lax.iota(jnp.int32, L)
    zi = jnp.zeros((L,), jnp.int32)
    negv = jnp.full((L,), NEG, jnp.float32)
    lane0 = iota == 0

    pltpu.sync_copy(x_hbm.at[pl.ds(src, CHUNK)], x_v)
    pltpu.sync_copy(x_hbm.at[pl.ds(src, CHUNK)], work_v)

    res_v = jnp.full((L,), NEG, jnp.float32)
    res_i = jnp.full((L,), BIG, jnp.int32)
    for r in range(K):
        bv = jnp.full((L,), NEG, jnp.float32)
        bi = jnp.full((L,), BIG, jnp.int32)
        for j in range(NV):
            v = work_v[pl.ds(L * j, L)]
            m = v > bv
            bv = jnp.where(m, v, bv)
            bi = jnp.where(m, iota + (L * j), bi)
        gv = _bfly(bv, jnp.maximum, iota)
        gi = _bfly(jnp.where(bv == gv, bi, BIG), jnp.minimum, iota)
        res_v = jnp.where(iota == r, gv, res_v)
        res_i = jnp.where(iota == r, gi, res_i)
        plsc.store_scatter(work_v, [gi], negv, mask=lane0)

    row_v[...] = res_v
    row_i[...] = res_i + base
    pltpu.sync_copy(row_v, cv_sh.at[pl.ds(s * L, L)])
    pltpu.sync_copy(row_i, ci_sh.at[pl.ds(s * L, L)])
    plsc.subcore_barrier()
    pltpu.sync_copy(cv_sh, cv_loc)
    pltpu.sync_copy(ci_sh, ci_loc)

    mv = jnp.full((L,), NEG, jnp.float32)
    for j in range(NS):
        mv = jnp.maximum(mv, cv_loc[pl.ds(L * j, L)])
    gmax = _bfly(mv, jnp.maximum, iota)

    @pl.when(s == 0)
    def _merge():
        resm = jnp.zeros((L,), jnp.int32)
        for r in range(K):
            bv = jnp.full((L,), NEG, jnp.float32)
            bi = jnp.full((L,), BIG, jnp.int32)
            bp = jnp.full((L,), BIG, jnp.int32)
            for j in range(NS):
                v = cv_loc[pl.ds(L * j, L)]
                gx = ci_loc[pl.ds(L * j, L)]
                m = v > bv
                bv = jnp.where(m, v, bv)
                bi = jnp.where(m, gx, bi)
                bp = jnp.where(m, iota + (L * j), bp)
            gv = _bfly(bv, jnp.maximum, iota)
            vm = bv == gv
            gi = _bfly(jnp.where(vm, bi, BIG), jnp.minimum, iota)
            resm = jnp.where(iota == r, gi, resm)
            gp = _bfly(jnp.where(vm & (bi == gi), bp, BIG), jnp.minimum, iota)
            plsc.store_scatter(cv_loc, [gp], negv, mask=lane0)
        row_t[...] = resm
        pltpu.sync_copy(row_t.at[pl.ds(0, K)], top_hbm.at[pl.ds(c * K, K)])

    acc = jnp.zeros((L,), jnp.float32)
    for j in range(NV):
        e = jnp.exp(x_v[pl.ds(L * j, L)] - gmax)
        acc = acc + e
        e_v[pl.ds(L * j, L)] = e
    row_s[...] = jnp.where(lane0, _bfly(acc, jnp.add, iota), 0.0)
    pltpu.sync_copy(row_s, sum_sh.at[pl.ds(s * L, L)])
    plsc.subcore_barrier()
    pltpu.sync_copy(sum_sh, sum_loc)
    sacc = jnp.zeros((L,), jnp.float32)
    for j in range(NS):
        sacc = sacc + sum_loc[pl.ds(L * j, L)]
    inv = 1.0 / _bfly(sacc, jnp.add, iota)
    for j in range(NV):
        e_v[pl.ds(L * j, L)] = e_v[pl.ds(L * j, L)] * inv
    pltpu.sync_copy(e_v, w_hbm.at[pl.ds(src, CHUNK)])


@jax.jit
def _run(x):
    kern = pl.kernel(
        _body,
        out_type=[
            jax.ShapeDtypeStruct((NC * N,), jnp.float32),
            jax.ShapeDtypeStruct((NC * K,), jnp.int32),
        ],
        mesh=plsc.VectorSubcoreMesh(
            core_axis_name="c", subcore_axis_name="s",
            num_cores=NC, num_subcores=NS),
        scratch_types=[
            pltpu.VMEM((CHUNK,), jnp.float32),
            pltpu.VMEM((CHUNK,), jnp.float32),
            pltpu.VMEM((CHUNK,), jnp.float32),
            pltpu.VMEM_SHARED((NCAND,), jnp.float32),
            pltpu.VMEM_SHARED((NCAND,), jnp.int32),
            pltpu.VMEM_SHARED((NCAND,), jnp.float32),
            pltpu.VMEM((NCAND,), jnp.float32),
            pltpu.VMEM((NCAND,), jnp.int32),
            pltpu.VMEM((NCAND,), jnp.float32),
            pltpu.VMEM((L,), jnp.float32),
            pltpu.VMEM((L,), jnp.int32),
            pltpu.VMEM((L,), jnp.float32),
            pltpu.VMEM((L,), jnp.int32),
        ],
        compiler_params=pltpu.CompilerParams(needs_layout_passes=False),
        name="softmax_top8_sc",
    )
    return kern(x)


def kernel(fc1_logits, fc2_logits):
    x = jnp.concatenate([fc1_logits, fc2_logits])
    w, top = _run(x)
    return w[:N], w[N:], top[:K], top[K:]
